# baseline (speedup 1.0000x reference)
import functools

import jax
import jax.numpy as jnp
import numpy as np
from jax import lax
from jax.experimental import pallas as pl
from jax.experimental.pallas import tpu as pltpu
from jax.experimental.pallas import tpu_sc as plsc

_ROWS = 128
_N = 32768
_L = 16
_NV = _N // _L
_Q = 4
_NVQ = _NV // _Q
_NQ = _N // _Q
_QSHIFT = 13
_SHIFTS = (0, 11, 22)
_MASKS = (0x7FF, 0x7FF, 0x3FF)
_RSIZE = (2048, 2048, 1024)
_RBITS = (11, 11, 10)
_NC = 2
_NS = 16
_ROWS_PER_W = _ROWS // (_NC * _NS)
_MININT = np.int32(-2147483648)


def _to_sortable(u):
    return u ^ (jnp.right_shift(u, 31) | _MININT)


def _from_sortable(u):
    return u ^ (jnp.right_shift(~u, 31) | _MININT)


def _digit(u, shift, mask):
    ub = plsc.bitcast(u, jnp.uint32)
    return ((ub >> shift) & jnp.uint32(mask)).astype(jnp.int32)


def _last_lane(v):
    return lax.squeeze(lax.slice(v, (_L - 1,), (_L,)), (0,))


def _sort_body(x_hbm, out_hbm, a_v, b_v, h0_v, h1_v, h2_v):
    wid = lax.axis_index("s") * _NC + lax.axis_index("c")
    hists = (h0_v, h1_v, h2_v)
    zeros = jnp.zeros((_L,), jnp.int32)

    def do_row(r, _):
        row = wid * _ROWS_PER_W + r
        pltpu.sync_copy(x_hbm.at[row], a_v)

        def zero_lo(j, _):
            sl = pl.ds(j * _L, _L)
            h0_v[sl] = zeros
            h1_v[sl] = zeros
            h2_v[sl] = zeros
            return 0

        def zero_hi(j, _):
            sl = pl.ds(j * _L, _L)
            h0_v[sl] = zeros
            h1_v[sl] = zeros
            return 0

        lax.fori_loop(0, _Q * 1024 // _L, zero_lo, 0, unroll=8)
        lax.fori_loop(_Q * 1024 // _L, _Q * 2048 // _L, zero_hi, 0,
                      unroll=8)

        def hist0(i, _):
            for q in range(_Q):
                sl = pl.ds((q * _NVQ + i) * _L, _L)
                u = plsc.bitcast(a_v[sl], jnp.int32)
                u = _to_sortable(u)
                a_v[sl] = plsc.bitcast(u, jnp.float32)
                d = _digit(u, _SHIFTS[0], _MASKS[0])
                cnt, last = plsc.scan_count(d)
                plsc.addupdate_scatter(h0_v, [d + np.int32(q * _RSIZE[0])],
                                       cnt, mask=last)
            return 0

        lax.fori_loop(0, _NVQ, hist0, 0)

        for p in range(3):
            src, dst = (a_v, b_v) if p % 2 == 0 else (b_v, a_v)
            shift, mask, hist = _SHIFTS[p], _MASKS[p], hists[p]
            rsz = _RSIZE[p]

            def prefix(j, carry, hist=hist, rsz=rsz):
                sl = [pl.ds(q * rsz + j * _L, _L) for q in range(_Q)]
                v = [hist[s] for s in sl]
                t = (v[0] + v[1]) + (v[2] + v[3])
                c = plsc.cumsum(t)
                excl = c - t + carry
                for q in range(_Q):
                    hist[sl[q]] = excl
                    if q < _Q - 1:
                        excl = excl + v[q]
                return carry + _last_lane(c)

            lax.fori_loop(0, rsz // _L, prefix, jnp.int32(0), unroll=2)

            def permute(i, _, src=src, dst=dst, shift=shift, mask=mask,
                        hist=hist, rsz=rsz, p=p):
                for q in range(_Q):
                    u = plsc.bitcast(src[pl.ds((q * _NVQ + i) * _L, _L)],
                                     jnp.int32)
                    d = _digit(u, shift, mask)
                    cnt, last = plsc.scan_count(d)
                    dq = d + np.int32(q * rsz)
                    base = plsc.load_gather(hist, [dq])
                    pos = base + cnt - 1
                    out = _from_sortable(u) if p == 2 else u
                    plsc.store_scatter(dst, [pos],
                                       plsc.bitcast(out, jnp.float32))
                    plsc.store_scatter(hist, [dq], base + cnt, mask=last)
                    if p < 2:
                        dn = _digit(u, _SHIFTS[p + 1], _MASKS[p + 1])
                        ub = plsc.bitcast(pos, jnp.uint32)
                        qn = ((ub >> _QSHIFT) << _RBITS[p + 1]).astype(
                            jnp.int32)
                        cn = qn + dn
                        cnt2, last2 = plsc.scan_count(cn)
                        plsc.addupdate_scatter(hists[p + 1], [cn], cnt2,
                                               mask=last2)
                return 0

            lax.fori_loop(0, _NVQ, permute, 0)

        pltpu.sync_copy(b_v, out_hbm.at[row])
        return 0

    lax.fori_loop(0, _ROWS_PER_W, do_row, 0)


@jax.jit
def kernel(x):
    mesh = plsc.VectorSubcoreMesh(
        core_axis_name="c", subcore_axis_name="s", num_cores=_NC,
        num_subcores=_NS)
    run = pl.kernel(
        _sort_body,
        out_type=jax.ShapeDtypeStruct((_ROWS, _N), jnp.float32),
        mesh=mesh,
        scratch_types=[
            pltpu.VMEM((_N,), jnp.float32),
            pltpu.VMEM((_N,), jnp.float32),
            pltpu.VMEM((_Q * 2048,), jnp.int32),
            pltpu.VMEM((_Q * 2048,), jnp.int32),
            pltpu.VMEM((_Q * 1024,), jnp.int32),
        ],
        compiler_params=pltpu.CompilerParams(needs_layout_passes=False),
    )
    return run(x)
